# baseline (speedup 1.0000x reference)
import jax
import jax.numpy as jnp
from jax.experimental import pallas as pl
from jax.experimental.pallas import tpu as pltpu

_PI = 3.141592653589793
_BLK = 6272

_S = (9.999791148949e-01, -1.666240153832e-01, 8.308849931241e-03,
      -1.926316995274e-04, 2.147049615597e-06)
_C = (9.999992107412e-01, -4.999942131496e-01, 4.165977758565e-02,
      -1.385878920428e-03, 2.420293205105e-05, -2.197292187089e-07)


def _sincos(v):
    t = v * v
    s = (((_S[4] * t + _S[3]) * t + _S[2]) * t + _S[1]) * t + _S[0]
    s = s * v
    c = ((((_C[5] * t + _C[4]) * t + _C[3]) * t + _C[2]) * t + _C[1]) * t + _C[0]
    return s, c


def _score_kernel(trip_ref, entp_ref, relp_ref, blk_ref, ent_hbm, rel_hbm,
                  out_ref, k_ref, hg_ref, rg_ref, cu_ref, su_ref,
                  sem_h, sem_r):
    b_sz = out_ref.shape[0]
    j = pl.program_id(0)

    @pl.when(j == 0)
    def _init():
        copies = []
        for b in range(b_sz):
            h = trip_ref[b, 0]
            r = trip_ref[b, 1]
            ch = pltpu.make_async_copy(
                ent_hbm.at[pl.ds(h, 1), :], hg_ref.at[pl.ds(b, 1), :],
                sem_h.at[b])
            cr = pltpu.make_async_copy(
                rel_hbm.at[pl.ds(r, 1), :], rg_ref.at[pl.ds(b, 1), :],
                sem_r.at[b])
            ch.start()
            cr.start()
            copies.append((ch, cr))
        me = jnp.max(jnp.abs(entp_ref[...]))
        mr = jnp.max(jnp.abs(relp_ref[...]))
        ke = _PI / me
        kr = _PI / mr
        k_ref[0, 0] = ke
        for ch, cr in copies:
            ch.wait()
            cr.wait()
        u = hg_ref[...] * ke + rg_ref[...] * kr
        cu_ref[...] = jnp.cos(u)
        su_ref[...] = jnp.sin(u)

    ke = k_ref[0, 0]
    v = blk_ref[...].T * ke
    s, c = _sincos(v)
    dn = (((1,), (0,)), ((), ()))
    out_ref[...] = (
        jax.lax.dot_general(cu_ref[...], s, dn,
                            preferred_element_type=jnp.float32)
        - jax.lax.dot_general(su_ref[...], c, dn,
                              preferred_element_type=jnp.float32))


def kernel(triples, ent_emb, rel_emb):
    batch = triples.shape[0]
    num_ent, dim = ent_emb.shape
    num_blk = (num_ent + _BLK - 1) // _BLK

    ent_packed = jnp.reshape(ent_emb, (num_ent * dim // 128, 128))
    rel_packed = jnp.reshape(rel_emb, (rel_emb.size // 128, 128))
    trip = triples.astype(jnp.int32)

    return pl.pallas_call(
        _score_kernel,
        grid=(num_blk,),
        in_specs=[
            pl.BlockSpec(memory_space=pltpu.MemorySpace.SMEM),
            pl.BlockSpec(ent_packed.shape, lambda j: (0, 0)),
            pl.BlockSpec(rel_packed.shape, lambda j: (0, 0)),
            pl.BlockSpec((_BLK, dim), lambda j: (j, 0)),
            pl.BlockSpec(memory_space=pltpu.MemorySpace.HBM),
            pl.BlockSpec(memory_space=pltpu.MemorySpace.HBM),
        ],
        out_specs=pl.BlockSpec((batch, _BLK), lambda j: (0, j)),
        out_shape=jax.ShapeDtypeStruct((batch, num_ent), jnp.float32),
        scratch_shapes=[
            pltpu.SMEM((1, 1), jnp.float32),
            pltpu.VMEM((batch, dim), jnp.float32),
            pltpu.VMEM((batch, dim), jnp.float32),
            pltpu.VMEM((batch, dim), jnp.float32),
            pltpu.VMEM((batch, dim), jnp.float32),
            pltpu.SemaphoreType.DMA((batch,)),
            pltpu.SemaphoreType.DMA((batch,)),
        ],
    )(trip, ent_packed, rel_packed, ent_emb, ent_emb, rel_emb)


# speedup vs baseline: 1.4223x; 1.4223x over previous
import jax
import jax.numpy as jnp
from jax.experimental import pallas as pl
from jax.experimental.pallas import tpu as pltpu

_PI = 3.141592653589793
_BLK = 6272

_S = (9.999791148949e-01, -1.666240153832e-01, 8.308849931241e-03,
      -1.926316995274e-04, 2.147049615597e-06)
_C = (9.999992107412e-01, -4.999942131496e-01, 4.165977758565e-02,
      -1.385878920428e-03, 2.420293205105e-05, -2.197292187089e-07)


def _sincos(v):
    t = v * v
    s = (((_S[4] * t + _S[3]) * t + _S[2]) * t + _S[1]) * t + _S[0]
    s = s * v
    c = ((((_C[5] * t + _C[4]) * t + _C[3]) * t + _C[2]) * t + _C[1]) * t + _C[0]
    return s, c


def _chunks(n):
    out = []
    base = 0
    while base < n:
        w = min(_BLK, n - base)
        out.append((base, w))
        base += w
    return out


def _score_kernel(trip_ref, ent_hbm, rel_hbm, out_hbm,
                  entT_ref, in0_ref, in1_ref, out0_ref, out1_ref, outt_ref,
                  relb_ref, hg_ref, rg_ref,
                  sem_in, sem_out, sem_rel, sem_h, sem_r):
    b_sz = out0_ref.shape[0]
    n = ent_hbm.shape[0]
    chunks = _chunks(n)
    inbufs = (in0_ref, in1_ref)
    outbufs = (out0_ref, out1_ref)

    gathers = []
    for b in range(b_sz):
        h = trip_ref[b, 0]
        r = trip_ref[b, 1]
        ch = pltpu.make_async_copy(
            ent_hbm.at[pl.ds(h, 1), :], hg_ref.at[pl.ds(b, 1), :],
            sem_h.at[b])
        cr = pltpu.make_async_copy(
            rel_hbm.at[pl.ds(r, 1), :], rg_ref.at[pl.ds(b, 1), :],
            sem_r.at[b])
        ch.start()
        cr.start()
        gathers.append((ch, cr))
    rel_cp = pltpu.make_async_copy(rel_hbm, relb_ref, sem_rel)
    rel_cp.start()

    def in_dma(j):
        base, w = chunks[j]
        return pltpu.make_async_copy(
            ent_hbm.at[pl.ds(base, w), :],
            inbufs[j % 2].at[pl.ds(0, w), :], sem_in.at[j])

    in_dmas = [in_dma(j) for j in range(len(chunks))]
    in_dmas[0].start()
    in_dmas[1].start()

    me = jnp.float32(0)
    for j, (base, w) in enumerate(chunks):
        in_dmas[j].wait()
        tb = inbufs[j % 2][pl.ds(0, w), :].T
        entT_ref[:, pl.ds(base, w)] = tb
        me = jnp.maximum(me, jnp.max(jnp.abs(tb)))
        if j + 2 < len(chunks):
            in_dmas[j + 2].start()

    rel_cp.wait()
    mr = jnp.max(jnp.abs(relb_ref[...]))
    ke = _PI / me
    kr = _PI / mr

    for ch, cr in gathers:
        ch.wait()
        cr.wait()
    u = hg_ref[...] * ke + rg_ref[...] * kr
    cu = jnp.cos(u)
    su = jnp.sin(u)

    dn = (((1,), (0,)), ((), ()))
    out_dmas = []
    for j, (base, w) in enumerate(chunks):
        if j >= 2:
            out_dmas[j - 2].wait()
        v = entT_ref[:, pl.ds(base, w)] * ke
        s, c = _sincos(v)
        res = (
            jax.lax.dot_general(cu, s, dn, preferred_element_type=jnp.float32)
            - jax.lax.dot_general(su, c, dn,
                                  preferred_element_type=jnp.float32))
        if w == _BLK:
            ob = outbufs[j % 2]
            ob[...] = res
            cp = pltpu.make_async_copy(
                ob, out_hbm.at[:, pl.ds(base, w)], sem_out.at[j])
        else:
            outt_ref[...] = res
            cp = pltpu.make_async_copy(
                outt_ref, out_hbm.at[:, pl.ds(base, w)], sem_out.at[j])
        cp.start()
        out_dmas.append(cp)
    out_dmas[-2].wait()
    out_dmas[-1].wait()


def kernel(triples, ent_emb, rel_emb):
    batch = triples.shape[0]
    num_ent, dim = ent_emb.shape
    n_chunk = (num_ent + _BLK - 1) // _BLK
    n_pad = n_chunk * _BLK
    trip = triples.astype(jnp.int32)

    return pl.pallas_call(
        _score_kernel,
        in_specs=[
            pl.BlockSpec(memory_space=pltpu.MemorySpace.SMEM),
            pl.BlockSpec(memory_space=pltpu.MemorySpace.HBM),
            pl.BlockSpec(memory_space=pltpu.MemorySpace.HBM),
        ],
        out_specs=pl.BlockSpec(memory_space=pltpu.MemorySpace.HBM),
        out_shape=jax.ShapeDtypeStruct((batch, num_ent), jnp.float32),
        scratch_shapes=[
            pltpu.VMEM((dim, n_pad), jnp.float32),
            pltpu.VMEM((_BLK, dim), jnp.float32),
            pltpu.VMEM((_BLK, dim), jnp.float32),
            pltpu.VMEM((batch, _BLK), jnp.float32),
            pltpu.VMEM((batch, _BLK), jnp.float32),
            pltpu.VMEM((batch, num_ent - (n_chunk - 1) * _BLK), jnp.float32),
            pltpu.VMEM(rel_emb.shape, jnp.float32),
            pltpu.VMEM((batch, dim), jnp.float32),
            pltpu.VMEM((batch, dim), jnp.float32),
            pltpu.SemaphoreType.DMA((n_chunk,)),
            pltpu.SemaphoreType.DMA((n_chunk,)),
            pltpu.SemaphoreType.DMA,
            pltpu.SemaphoreType.DMA((batch,)),
            pltpu.SemaphoreType.DMA((batch,)),
        ],
    )(trip, ent_emb, rel_emb)


# speedup vs baseline: 1.4568x; 1.0242x over previous
import jax
import jax.numpy as jnp
from jax.experimental import pallas as pl
from jax.experimental.pallas import tpu as pltpu

_PI = 3.141592653589793
_BLK = 6272

_S = (9.999791148949e-01, -1.666240153832e-01, 8.308849931241e-03,
      -1.926316995274e-04, 2.147049615597e-06)
_C = (9.999992107412e-01, -4.999942131496e-01, 4.165977758565e-02,
      -1.385878920428e-03, 2.420293205105e-05, -2.197292187089e-07)


def _sincos(v):
    t = v * v
    s = (((_S[4] * t + _S[3]) * t + _S[2]) * t + _S[1]) * t + _S[0]
    s = s * v
    c = ((((_C[5] * t + _C[4]) * t + _C[3]) * t + _C[2]) * t + _C[1]) * t + _C[0]
    return s, c


def _chunks(n):
    out = []
    base = 0
    while base < n:
        w = min(_BLK, n - base)
        out.append((base, w))
        base += w
    return out


def _score_kernel(trip_ref, ent_hbm, rel_hbm, out_hbm,
                  entT_ref, in0_ref, in1_ref, out0_ref, out1_ref, outt_ref,
                  relb_ref, hg_ref, rg_ref,
                  sem_in, sem_out, sem_rel, sem_h, sem_r):
    b_sz = out0_ref.shape[0]
    n = ent_hbm.shape[0]
    chunks = _chunks(n)
    inbufs = (in0_ref, in1_ref)
    outbufs = (out0_ref, out1_ref)

    gathers = []
    for b in range(b_sz):
        h = trip_ref[b, 0]
        r = trip_ref[b, 1]
        ch = pltpu.make_async_copy(
            ent_hbm.at[pl.ds(h, 1), :], hg_ref.at[pl.ds(b, 1), :],
            sem_h.at[b])
        cr = pltpu.make_async_copy(
            rel_hbm.at[pl.ds(r, 1), :], rg_ref.at[pl.ds(b, 1), :],
            sem_r.at[b])
        ch.start()
        cr.start()
        gathers.append((ch, cr))
    rel_cp = pltpu.make_async_copy(rel_hbm, relb_ref, sem_rel)
    rel_cp.start()

    def in_dma(j):
        base, w = chunks[j]
        return pltpu.make_async_copy(
            ent_hbm.at[pl.ds(base, w), :],
            inbufs[j % 2].at[pl.ds(0, w), :], sem_in.at[j])

    in_dmas = [in_dma(j) for j in range(len(chunks))]
    in_dmas[0].start()
    in_dmas[1].start()

    me = jnp.float32(0)
    for j, (base, w) in enumerate(chunks):
        in_dmas[j].wait()
        tb = inbufs[j % 2][pl.ds(0, w), :].T
        entT_ref[:, pl.ds(base, w)] = tb
        me = jnp.maximum(me, jnp.max(jnp.abs(tb)))
        if j + 2 < len(chunks):
            in_dmas[j + 2].start()

    rel_cp.wait()
    mr = jnp.max(jnp.abs(relb_ref[...]))
    ke = _PI / me
    kr = _PI / mr

    for ch, cr in gathers:
        ch.wait()
        cr.wait()
    u = hg_ref[...] * ke + rg_ref[...] * kr
    cu = jnp.cos(u)
    su = jnp.sin(u)

    dn = (((1,), (0,)), ((), ()))
    for j, (base, w) in enumerate(chunks):
        v = entT_ref[:, pl.ds(base, w)] * ke
        s, c = _sincos(v)
        out_hbm[:, pl.ds(base, w)] = (
            jax.lax.dot_general(cu, s, dn, preferred_element_type=jnp.float32)
            - jax.lax.dot_general(su, c, dn,
                                  preferred_element_type=jnp.float32))


def kernel(triples, ent_emb, rel_emb):
    batch = triples.shape[0]
    num_ent, dim = ent_emb.shape
    n_chunk = (num_ent + _BLK - 1) // _BLK
    n_pad = n_chunk * _BLK
    trip = triples.astype(jnp.int32)

    return pl.pallas_call(
        _score_kernel,
        in_specs=[
            pl.BlockSpec(memory_space=pltpu.MemorySpace.SMEM),
            pl.BlockSpec(memory_space=pltpu.MemorySpace.HBM),
            pl.BlockSpec(memory_space=pltpu.MemorySpace.HBM),
        ],
        out_specs=pl.BlockSpec(memory_space=pltpu.MemorySpace.VMEM),
        out_shape=jax.ShapeDtypeStruct((batch, num_ent), jnp.float32),
        scratch_shapes=[
            pltpu.VMEM((dim, n_pad), jnp.float32),
            pltpu.VMEM((_BLK, dim), jnp.float32),
            pltpu.VMEM((_BLK, dim), jnp.float32),
            pltpu.VMEM((batch, _BLK), jnp.float32),
            pltpu.VMEM((batch, _BLK), jnp.float32),
            pltpu.VMEM((batch, num_ent - (n_chunk - 1) * _BLK), jnp.float32),
            pltpu.VMEM(rel_emb.shape, jnp.float32),
            pltpu.VMEM((batch, dim), jnp.float32),
            pltpu.VMEM((batch, dim), jnp.float32),
            pltpu.SemaphoreType.DMA((n_chunk,)),
            pltpu.SemaphoreType.DMA((n_chunk,)),
            pltpu.SemaphoreType.DMA,
            pltpu.SemaphoreType.DMA((batch,)),
            pltpu.SemaphoreType.DMA((batch,)),
        ],
    )(trip, ent_emb, rel_emb)


# speedup vs baseline: 2.0386x; 1.3994x over previous
import jax
import jax.numpy as jnp
from jax.experimental import pallas as pl
from jax.experimental.pallas import tpu as pltpu

_PI = 3.141592653589793
_BLK = 6272

_S = (9.999791148949e-01, -1.666240153832e-01, 8.308849931241e-03,
      -1.926316995274e-04, 2.147049615597e-06)
_C = (9.999992107412e-01, -4.999942131496e-01, 4.165977758565e-02,
      -1.385878920428e-03, 2.420293205105e-05, -2.197292187089e-07)


def _sincos(v):
    t = v * v
    s = (((_S[4] * t + _S[3]) * t + _S[2]) * t + _S[1]) * t + _S[0]
    s = s * v
    c = ((((_C[5] * t + _C[4]) * t + _C[3]) * t + _C[2]) * t + _C[1]) * t + _C[0]
    return s, c


def _score_kernel(trip_ref, entT_ref, ent_hbm, rel_hbm, out_ref,
                  k_ref, relb_ref, hg_ref, rg_ref, cu_ref, su_ref,
                  sem_rel, sem_h, sem_r):
    b_sz = out_ref.shape[0]
    n = entT_ref.shape[1]
    n_blk = pl.num_programs(0)
    j = pl.program_id(0)
    tail_w = n - (n_blk - 1) * _BLK

    @pl.when(j == 0)
    def _init():
        copies = []
        for b in range(b_sz):
            h = trip_ref[b, 0]
            r = trip_ref[b, 1]
            ch = pltpu.make_async_copy(
                ent_hbm.at[pl.ds(h, 1), :], hg_ref.at[pl.ds(b, 1), :],
                sem_h.at[b])
            cr = pltpu.make_async_copy(
                rel_hbm.at[pl.ds(r, 1), :], rg_ref.at[pl.ds(b, 1), :],
                sem_r.at[b])
            ch.start()
            cr.start()
            copies.append((ch, cr))
        rel_cp = pltpu.make_async_copy(rel_hbm, relb_ref, sem_rel)
        rel_cp.start()

        me = jnp.float32(0)
        base = 0
        while base < n:
            w = min(_BLK, n - base)
            me = jnp.maximum(
                me, jnp.max(jnp.abs(entT_ref[:, pl.ds(base, w)])))
            base += w
        rel_cp.wait()
        mr = jnp.max(jnp.abs(relb_ref[...]))
        ke = _PI / me
        kr = _PI / mr
        k_ref[0, 0] = ke

        for ch, cr in copies:
            ch.wait()
            cr.wait()
        u = hg_ref[...] * ke + rg_ref[...] * kr
        cu_ref[...] = jnp.cos(u)
        su_ref[...] = jnp.sin(u)

    ke = k_ref[0, 0]
    cu = cu_ref[...]
    su = su_ref[...]
    dn = (((1,), (0,)), ((), ()))

    def _emit(base, w):
        v = entT_ref[:, pl.ds(base, w)] * ke
        s, c = _sincos(v)
        out_ref[:, pl.ds(0, w)] = (
            jax.lax.dot_general(cu, s, dn, preferred_element_type=jnp.float32)
            - jax.lax.dot_general(su, c, dn,
                                  preferred_element_type=jnp.float32))

    @pl.when(j < n_blk - 1)
    def _full():
        _emit(j * _BLK, _BLK)

    @pl.when(j == n_blk - 1)
    def _tail():
        _emit((n_blk - 1) * _BLK, tail_w)


def kernel(triples, ent_emb, rel_emb):
    batch = triples.shape[0]
    num_ent, dim = ent_emb.shape
    n_blk = (num_ent + _BLK - 1) // _BLK

    entT = ent_emb.T
    trip = triples.astype(jnp.int32)

    return pl.pallas_call(
        _score_kernel,
        grid=(n_blk,),
        in_specs=[
            pl.BlockSpec(memory_space=pltpu.MemorySpace.SMEM),
            pl.BlockSpec((dim, num_ent), lambda j: (0, 0)),
            pl.BlockSpec(memory_space=pltpu.MemorySpace.HBM),
            pl.BlockSpec(memory_space=pltpu.MemorySpace.HBM),
        ],
        out_specs=pl.BlockSpec((batch, _BLK), lambda j: (0, j)),
        out_shape=jax.ShapeDtypeStruct((batch, num_ent), jnp.float32),
        scratch_shapes=[
            pltpu.SMEM((1, 1), jnp.float32),
            pltpu.VMEM(rel_emb.shape, jnp.float32),
            pltpu.VMEM((batch, dim), jnp.float32),
            pltpu.VMEM((batch, dim), jnp.float32),
            pltpu.VMEM((batch, dim), jnp.float32),
            pltpu.VMEM((batch, dim), jnp.float32),
            pltpu.SemaphoreType.DMA,
            pltpu.SemaphoreType.DMA((batch,)),
            pltpu.SemaphoreType.DMA((batch,)),
        ],
    )(trip, entT, ent_emb, rel_emb)


# speedup vs baseline: 2.0705x; 1.0156x over previous
import jax
import jax.numpy as jnp
from jax.experimental import pallas as pl
from jax.experimental.pallas import tpu as pltpu

_PI = 3.141592653589793
_CHUNK = 6272

_S = (9.999791148949e-01, -1.666240153832e-01, 8.308849931241e-03,
      -1.926316995274e-04, 2.147049615597e-06)
_C = (9.999992107412e-01, -4.999942131496e-01, 4.165977758565e-02,
      -1.385878920428e-03, 2.420293205105e-05, -2.197292187089e-07)


def _sincos(v):
    t = v * v
    s = (((_S[4] * t + _S[3]) * t + _S[2]) * t + _S[1]) * t + _S[0]
    s = s * v
    c = ((((_C[5] * t + _C[4]) * t + _C[3]) * t + _C[2]) * t + _C[1]) * t + _C[0]
    return s, c


def _chunks(n):
    out = []
    base = 0
    while base < n:
        w = min(_CHUNK, n - base)
        out.append((base, w))
        base += w
    return out


def _score_kernel(trip_ref, entT_ref, relT_ref, ent_hbm, rel_hbm, out_hbm,
                  hg_ref, rg_ref, out0_ref, out1_ref, outt_ref,
                  sem_h, sem_r, sem_out):
    b_sz = out_hbm.shape[0]
    n = out_hbm.shape[1]
    outbufs = (out0_ref, out1_ref)

    copies = []
    for b in range(b_sz):
        h = trip_ref[b, 0]
        r = trip_ref[b, 1]
        ch = pltpu.make_async_copy(
            ent_hbm.at[pl.ds(h, 1), :], hg_ref.at[pl.ds(b, 1), :],
            sem_h.at[b])
        cr = pltpu.make_async_copy(
            rel_hbm.at[pl.ds(r, 1), :], rg_ref.at[pl.ds(b, 1), :],
            sem_r.at[b])
        ch.start()
        cr.start()
        copies.append((ch, cr))

    me = jnp.float32(0)
    for base, w in _chunks(n):
        me = jnp.maximum(me, jnp.max(jnp.abs(entT_ref[:, pl.ds(base, w)])))
    mr = jnp.max(jnp.abs(relT_ref[...]))
    ke = _PI / me
    kr = _PI / mr

    for ch, cr in copies:
        ch.wait()
        cr.wait()
    u = hg_ref[...] * ke + rg_ref[...] * kr
    cu = jnp.cos(u)
    su = jnp.sin(u)

    dn = (((1,), (0,)), ((), ()))
    out_dmas = []
    for j, (base, w) in enumerate(_chunks(n)):
        if j >= 2:
            out_dmas[j - 2].wait()
        v = entT_ref[:, pl.ds(base, w)] * ke
        s, c = _sincos(v)
        res = (
            jax.lax.dot_general(cu, s, dn, preferred_element_type=jnp.float32)
            - jax.lax.dot_general(su, c, dn,
                                  preferred_element_type=jnp.float32))
        ob = outbufs[j % 2] if w == _CHUNK else outt_ref
        ob[...] = res
        cp = pltpu.make_async_copy(
            ob, out_hbm.at[:, pl.ds(base, w)], sem_out.at[j])
        cp.start()
        out_dmas.append(cp)
    out_dmas[-2].wait()
    out_dmas[-1].wait()


def kernel(triples, ent_emb, rel_emb):
    batch = triples.shape[0]
    num_ent, dim = ent_emb.shape

    entT = ent_emb.T
    relT = rel_emb.T
    trip = triples.astype(jnp.int32)

    return pl.pallas_call(
        _score_kernel,
        in_specs=[
            pl.BlockSpec(memory_space=pltpu.MemorySpace.SMEM),
            pl.BlockSpec(memory_space=pltpu.MemorySpace.VMEM),
            pl.BlockSpec(memory_space=pltpu.MemorySpace.VMEM),
            pl.BlockSpec(memory_space=pltpu.MemorySpace.HBM),
            pl.BlockSpec(memory_space=pltpu.MemorySpace.HBM),
        ],
        out_specs=pl.BlockSpec(memory_space=pltpu.MemorySpace.HBM),
        out_shape=jax.ShapeDtypeStruct((batch, num_ent), jnp.float32),
        scratch_shapes=[
            pltpu.VMEM((batch, dim), jnp.float32),
            pltpu.VMEM((batch, dim), jnp.float32),
            pltpu.VMEM((batch, _CHUNK), jnp.float32),
            pltpu.VMEM((batch, _CHUNK), jnp.float32),
            pltpu.VMEM((batch, num_ent - (num_ent // _CHUNK) * _CHUNK),
                       jnp.float32),
            pltpu.SemaphoreType.DMA((batch,)),
            pltpu.SemaphoreType.DMA((batch,)),
            pltpu.SemaphoreType.DMA(((num_ent + _CHUNK - 1) // _CHUNK,)),
        ],
    )(trip, entT, relT, ent_emb, rel_emb)


# speedup vs baseline: 2.1302x; 1.0288x over previous
import jax
import jax.numpy as jnp
from jax.experimental import pallas as pl
from jax.experimental.pallas import tpu as pltpu

_PI = 3.141592653589793
_CHUNK = 6272

_S = (9.999791148949e-01, -1.666240153832e-01, 8.308849931241e-03,
      -1.926316995274e-04, 2.147049615597e-06)
_C = (9.999992107412e-01, -4.999942131496e-01, 4.165977758565e-02,
      -1.385878920428e-03, 2.420293205105e-05, -2.197292187089e-07)


def _sincos(v):
    t = v * v
    s = (((_S[4] * t + _S[3]) * t + _S[2]) * t + _S[1]) * t + _S[0]
    s = s * v
    c = ((((_C[5] * t + _C[4]) * t + _C[3]) * t + _C[2]) * t + _C[1]) * t + _C[0]
    return s, c


def _chunks(n):
    out = []
    base = 0
    while base < n:
        w = min(_CHUNK, n - base)
        out.append((base, w))
        base += w
    return out


def _score_kernel(trip_ref, entT_ref, relT_ref, ent_hbm, rel_hbm, out_ref,
                  hg_ref, rg_ref, sem_h, sem_r):
    b_sz = out_ref.shape[0]
    n = out_ref.shape[1]

    copies = []
    for b in range(b_sz):
        h = trip_ref[b, 0]
        r = trip_ref[b, 1]
        ch = pltpu.make_async_copy(
            ent_hbm.at[pl.ds(h, 1), :], hg_ref.at[pl.ds(b, 1), :],
            sem_h.at[b])
        cr = pltpu.make_async_copy(
            rel_hbm.at[pl.ds(r, 1), :], rg_ref.at[pl.ds(b, 1), :],
            sem_r.at[b])
        ch.start()
        cr.start()
        copies.append((ch, cr))

    me = jnp.float32(0)
    for base, w in _chunks(n):
        me = jnp.maximum(me, jnp.max(jnp.abs(entT_ref[:, pl.ds(base, w)])))
    mr = jnp.max(jnp.abs(relT_ref[...]))
    ke = _PI / me
    kr = _PI / mr

    for ch, cr in copies:
        ch.wait()
        cr.wait()
    u = hg_ref[...] * ke + rg_ref[...] * kr
    cu = jnp.cos(u)
    su = jnp.sin(u)

    dn = (((1,), (0,)), ((), ()))
    for base, w in _chunks(n):
        v = entT_ref[:, pl.ds(base, w)] * ke
        s, c = _sincos(v)
        out_ref[:, pl.ds(base, w)] = (
            jax.lax.dot_general(cu, s, dn, preferred_element_type=jnp.float32)
            - jax.lax.dot_general(su, c, dn,
                                  preferred_element_type=jnp.float32))


def kernel(triples, ent_emb, rel_emb):
    batch = triples.shape[0]
    num_ent, dim = ent_emb.shape

    entT = ent_emb.T
    relT = rel_emb.T
    trip = triples.astype(jnp.int32)

    return pl.pallas_call(
        _score_kernel,
        in_specs=[
            pl.BlockSpec(memory_space=pltpu.MemorySpace.SMEM),
            pl.BlockSpec(memory_space=pltpu.MemorySpace.VMEM),
            pl.BlockSpec(memory_space=pltpu.MemorySpace.VMEM),
            pl.BlockSpec(memory_space=pltpu.MemorySpace.HBM),
            pl.BlockSpec(memory_space=pltpu.MemorySpace.HBM),
        ],
        out_specs=pl.BlockSpec(memory_space=pltpu.MemorySpace.VMEM),
        out_shape=jax.ShapeDtypeStruct((batch, num_ent), jnp.float32),
        scratch_shapes=[
            pltpu.VMEM((batch, dim), jnp.float32),
            pltpu.VMEM((batch, dim), jnp.float32),
            pltpu.SemaphoreType.DMA((batch,)),
            pltpu.SemaphoreType.DMA((batch,)),
        ],
    )(trip, entT, relT, ent_emb, rel_emb)
